# baseline (speedup 1.0000x reference)
import functools

import jax
import jax.numpy as jnp
from jax import lax
from jax.experimental import pallas as pl
from jax.experimental.pallas import tpu as pltpu
from jax.experimental.pallas import tpu_sc as plsc

N = 100000
C = 128
G = 64
GP = G + 8
NC = 2
NS = 16
NW = NC * NS
K = 160
KH = K // 2
TCH = N // K
FULL = TCH % NW
CPW = TCH // NW
L = 16
CJ = C // L
U = 4


def _body(feat, batch3, wflat, b16, out, acc, idx_v, fbuf, wv, bv, z8,
          flbuf, flidx, rseg, semf, semi):
    cid = lax.axis_index("c")
    sid = lax.axis_index("s")
    wid = cid * NS + sid

    pltpu.sync_copy(wflat, wv)
    pltpu.sync_copy(b16, bv)

    zero = jnp.zeros((L,), jnp.float32)
    lanes = lax.iota(jnp.int32, L)

    for r in range(8):
        for j in range(CJ):
            z8[r, pl.ds(L * j, L)] = zero

    @pl.when(sid < GP // 8)
    def _():
        pltpu.sync_copy(z8, acc.at[pl.ds(sid * 8, 8)])

    for r in range(16):
        for j in range(CJ):
            flbuf[r, pl.ds(L * j, L)] = zero
    rseg[pl.ds(0, L)] = jnp.full((L,), G, jnp.int32)

    plsc.subcore_barrier()

    bvec = bv[...]
    wregs = [wv[pl.ds(L * j, L)] for j in range(CJ)]
    nch = jnp.where(wid < FULL, CPW + 1, CPW)
    start = wid * CPW + jnp.minimum(wid, FULL)

    bfly = [lanes ^ sh for sh in (8, 4, 2, 1)]
    zidx = lanes * 0
    dnums = lax.GatherDimensionNumbers(
        offset_dims=(), collapsed_slice_dims=(0,), start_index_map=(0,)
    )

    def take16(x, idx):
        return lax.gather(
            x,
            idx[:, None],
            dnums,
            slice_sizes=(1,),
            mode=lax.GatherScatterMode.PROMISE_IN_BOUNDS,
        )

    def lane_sum_splat(x):
        for idx in bfly:
            x = x + take16(x, idx)
        return x

    def row_weight(fr):
        m = [fr[j] * wregs[j] for j in range(CJ)]
        while len(m) > 1:
            m = [m[2 * i] + m[2 * i + 1] for i in range(len(m) // 2)]
        s = lane_sum_splat(m[0])
        return 1.0 / (1.0 + jnp.exp(-(bvec + s)))

    def issue_fetch(c, p):
        pltpu.async_copy(feat.at[pl.ds(c * K, K)], fbuf.at[p], semf.at[p])
        pltpu.async_copy(batch3.at[c], idx_v.at[p], semi.at[p])

    def wait_fetch(p):
        pltpu.make_async_copy(feat.at[pl.ds(0, K)], fbuf.at[p], semf.at[p]).wait()
        pltpu.make_async_copy(batch3.at[0], idx_v.at[p], semi.at[p]).wait()

    def flush_open_run():
        cur = rseg[pl.ds(0, L)]
        flidx[pl.ds(0, L)] = jnp.where(lanes == 0, cur, G + (lanes & 7))
        pltpu.sync_copy(flbuf, acc.at[flidx], add=True)
        for j in range(CJ):
            flbuf[0, pl.ds(L * j, L)] = zero

    def process(c, t, p):
        wait_fetch(p)
        fb = fbuf.at[p]

        iv_first = idx_v[p, 0, pl.ds(0, L)]
        iv_last = idx_v[p, 1, pl.ds(KH - L, L)]
        fs = take16(iv_first, zidx)
        first = iv_first[0]
        last = iv_last[L - 1]
        cur = rseg[pl.ds(0, L)][0]
        uni = first == last
        same = first == cur

        @pl.when(jnp.logical_not(uni & same))
        def _():
            flush_open_run()

        @pl.when(uni)
        def _():
            rseg[pl.ds(0, L)] = fs

            def row_body(rq, racc):
                racc = list(racc)
                for i in range(U):
                    r = rq * U + i
                    fr = [fb[r, pl.ds(L * j, L)] for j in range(CJ)]
                    wgt = row_weight(fr)
                    for j in range(CJ):
                        racc[j] = racc[j] + fr[j] * wgt
                return tuple(racc)

            racc0 = tuple(flbuf[0, pl.ds(L * j, L)] for j in range(CJ))
            racc = lax.fori_loop(0, K // U, row_body, racc0)
            for j in range(CJ):
                flbuf[0, pl.ds(L * j, L)] = racc[j]

        @pl.when(jnp.logical_not(uni))
        def _():
            rseg[pl.ds(0, L)] = jnp.full((L,), G, jnp.int32)

            def row_body(rq, rc):
                for i in range(U):
                    r = rq * U + i
                    fr = [fb[r, pl.ds(L * j, L)] for j in range(CJ)]
                    wgt = row_weight(fr)
                    for j in range(CJ):
                        fb[r, pl.ds(L * j, L)] = fr[j] * wgt
                return rc

            lax.fori_loop(0, K // U, row_body, 0)
            for h in range(2):
                pltpu.sync_copy(fb.at[pl.ds(h * KH, KH)], acc.at[idx_v.at[p, h]],
                                add=True)

        @pl.when(t + 2 < nch)
        def _():
            issue_fetch(c + 2, p)

    issue_fetch(start, 0)
    issue_fetch(start + 1, 1)

    def chunk_body(t, carry):
        c = start + t

        @pl.when(t % 2 == 0)
        def _():
            process(c, t, 0)

        @pl.when(t % 2 == 1)
        def _():
            process(c, t, 1)

        return carry

    lax.fori_loop(0, nch, chunk_body, 0)
    flush_open_run()
    plsc.subcore_barrier()

    @pl.when(sid == 0)
    def _():
        pltpu.sync_copy(acc.at[pl.ds(0, G)], out.at[cid])


@jax.jit
def _pooling(features, batch3, wflat, b16):
    mesh = plsc.VectorSubcoreMesh(core_axis_name="c", subcore_axis_name="s")
    kfn = functools.partial(
        pl.kernel,
        mesh=mesh,
        out_type=jax.ShapeDtypeStruct((NC, G, C), jnp.float32),
        scratch_types=[
            pltpu.VMEM_SHARED((GP, C), jnp.float32),
            pltpu.VMEM((2, 2, KH), jnp.int32),
            pltpu.VMEM((2, K, C), jnp.float32),
            pltpu.VMEM((C,), jnp.float32),
            pltpu.VMEM((L,), jnp.float32),
            pltpu.VMEM((8, C), jnp.float32),
            pltpu.VMEM((16, C), jnp.float32),
            pltpu.VMEM((L,), jnp.int32),
            pltpu.VMEM((L,), jnp.int32),
            pltpu.SemaphoreType.DMA((2,)),
            pltpu.SemaphoreType.DMA((2,)),
        ],
    )(_body)
    return kfn(features, batch3, wflat, b16)


def kernel(features, batch, W, b):
    batch3 = batch.astype(jnp.int32).reshape(TCH, 2, KH)
    wflat = W.reshape(C).astype(jnp.float32)
    b16 = jnp.broadcast_to(b.reshape(()).astype(jnp.float32), (L,))
    partials = _pooling(features, batch3, wflat, b16)
    return partials[0] + partials[1]


# speedup vs baseline: 1.0145x; 1.0145x over previous
import functools

import jax
import jax.numpy as jnp
from jax import lax
from jax.experimental import pallas as pl
from jax.experimental.pallas import tpu as pltpu
from jax.experimental.pallas import tpu_sc as plsc

N = 100000
C = 128
G = 64
NC = 2
NS = 16
NW = NC * NS
K = 160
KH = K // 2
TCH = N // K
SCH = 375
FULL = SCH % NW
CPW = SCH // NW
L = 16
CJ = C // L
U = 4

TB = 400
TC0 = SCH * K // TB
TGRID = (N - SCH * K) // TB
NBB = N // TB


def _sc_body(feat, batch3, wflat, b16, out, acc, idx_v, idx_s, fbuf, sbuf,
             wv, bv, z8, semf, semi, sems):
    cid = lax.axis_index("c")
    sid = lax.axis_index("s")
    wid = cid * NS + sid

    pltpu.sync_copy(wflat, wv)
    pltpu.sync_copy(b16, bv)

    zero = jnp.zeros((L,), jnp.float32)
    for r in range(8):
        for j in range(CJ):
            z8[r, pl.ds(L * j, L)] = zero

    @pl.when(sid < 8)
    def _():
        pltpu.sync_copy(z8, acc.at[pl.ds(sid * 8, 8)])

    plsc.subcore_barrier()

    bvec = bv[...]
    wregs = [wv[pl.ds(L * j, L)] for j in range(CJ)]
    nch = jnp.where(wid < FULL, CPW + 1, CPW)
    start = wid * CPW + jnp.minimum(wid, FULL)

    lanes = lax.iota(jnp.int32, L)
    bfly = [lanes ^ sh for sh in (8, 4, 2, 1)]
    dnums = lax.GatherDimensionNumbers(
        offset_dims=(), collapsed_slice_dims=(0,), start_index_map=(0,)
    )

    def take16(x, idx):
        return lax.gather(
            x,
            idx[:, None],
            dnums,
            slice_sizes=(1,),
            mode=lax.GatherScatterMode.PROMISE_IN_BOUNDS,
        )

    def lane_sum_splat(x):
        for idx in bfly:
            x = x + take16(x, idx)
        return x

    def issue_fetch(c, p):
        pltpu.async_copy(feat.at[pl.ds(c * K, K)], fbuf.at[p], semf.at[p])
        pltpu.async_copy(batch3.at[c], idx_v.at[p], semi.at[p])

    def wait_fetch(p):
        pltpu.make_async_copy(feat.at[pl.ds(0, K)], fbuf.at[p], semf.at[p]).wait()
        pltpu.make_async_copy(batch3.at[0], idx_v.at[p], semi.at[p]).wait()

    def issue_scatter(p):
        for h in range(2):
            pltpu.async_copy(
                sbuf.at[p, pl.ds(h * KH, KH)],
                acc.at[idx_s.at[p, h]],
                sems.at[p],
                add=True,
            )

    def wait_scatter(p):
        for h in range(2):
            pltpu.make_async_copy(
                sbuf.at[p, pl.ds(h * KH, KH)], acc.at[idx_s.at[p, h]], sems.at[p]
            ).wait()

    def process(c, t, p):
        wait_fetch(p)

        @pl.when(t >= 2)
        def _():
            wait_scatter(p)

        for v in range(K // L):
            idx_s[p, v // (KH // L), pl.ds((v % (KH // L)) * L, L)] = idx_v[
                p, v // (KH // L), pl.ds((v % (KH // L)) * L, L)
            ]

        fb = fbuf.at[p]
        sb = sbuf.at[p]

        def scale_row(r):
            fr = [fb[r, pl.ds(L * j, L)] for j in range(CJ)]
            m = [fr[j] * wregs[j] for j in range(CJ)]
            while len(m) > 1:
                m = [m[2 * i] + m[2 * i + 1] for i in range(len(m) // 2)]
            s = lane_sum_splat(m[0])
            t_ = bvec + s
            wgt = 1.0 / (1.0 + jnp.exp(-t_))
            for j in range(CJ):
                sb[r, pl.ds(L * j, L)] = fr[j] * wgt

        def row_body(rq, rc):
            for i in range(U):
                scale_row(rq * U + i)
            return rc

        lax.fori_loop(0, K // U, row_body, 0)
        issue_scatter(p)

        @pl.when(t + 2 < nch)
        def _():
            issue_fetch(c + 2, p)

    issue_fetch(start, 0)
    issue_fetch(start + 1, 1)

    def chunk_body(t, carry):
        c = start + t

        @pl.when(t % 2 == 0)
        def _():
            process(c, t, 0)

        @pl.when(t % 2 == 1)
        def _():
            process(c, t, 1)

        return carry

    lax.fori_loop(0, nch, chunk_body, 0)
    wait_scatter(0)
    wait_scatter(1)
    plsc.subcore_barrier()

    @pl.when(sid == 0)
    def _():
        pltpu.sync_copy(acc, out.at[cid])


def _tc_body(feat_ref, batch_ref, w_ref, b_ref, out_ref):
    i = pl.program_id(0)
    blk = feat_ref[...]
    s = jnp.dot(blk, w_ref[...], preferred_element_type=jnp.float32)
    wgt = jax.nn.sigmoid(s + b_ref[0, 0])
    weighted = blk * wgt
    ids = batch_ref[0, 0, :]
    seg = lax.broadcasted_iota(jnp.int32, (G, TB), 0)
    onehot = (ids[None, :] == seg).astype(jnp.float32)
    part = jnp.dot(onehot, weighted, preferred_element_type=jnp.float32)

    @pl.when(i == 0)
    def _():
        out_ref[...] = part

    @pl.when(i > 0)
    def _():
        out_ref[...] += part


@jax.jit
def _pooling(features, batch3, batchb, wmat, b11, wflat, b16):
    mesh = plsc.VectorSubcoreMesh(core_axis_name="c", subcore_axis_name="s")
    sc_fn = functools.partial(
        pl.kernel,
        mesh=mesh,
        out_type=jax.ShapeDtypeStruct((NC, G, C), jnp.float32),
        scratch_types=[
            pltpu.VMEM_SHARED((G, C), jnp.float32),
            pltpu.VMEM((2, 2, KH), jnp.int32),
            pltpu.VMEM((2, 2, KH), jnp.int32),
            pltpu.VMEM((2, K, C), jnp.float32),
            pltpu.VMEM((2, K, C), jnp.float32),
            pltpu.VMEM((C,), jnp.float32),
            pltpu.VMEM((L,), jnp.float32),
            pltpu.VMEM((8, C), jnp.float32),
            pltpu.SemaphoreType.DMA((2,)),
            pltpu.SemaphoreType.DMA((2,)),
            pltpu.SemaphoreType.DMA((2,)),
        ],
    )(_sc_body)
    sc_part = sc_fn(features, batch3, wflat, b16)

    tc_part = pl.pallas_call(
        _tc_body,
        grid=(TGRID,),
        in_specs=[
            pl.BlockSpec((TB, C), lambda i: (TC0 + i, 0)),
            pl.BlockSpec((1, 1, TB), lambda i: (TC0 + i, 0, 0)),
            pl.BlockSpec((C, 1), lambda i: (0, 0)),
            pl.BlockSpec((1, 1), lambda i: (0, 0)),
        ],
        out_specs=pl.BlockSpec((G, C), lambda i: (0, 0)),
        out_shape=jax.ShapeDtypeStruct((G, C), jnp.float32),
    )(features, batchb, wmat, b11)

    return sc_part[0] + sc_part[1] + tc_part


def kernel(features, batch, W, b):
    bi = batch.astype(jnp.int32)
    batch3 = bi.reshape(TCH, 2, KH)
    batchb = bi.reshape(NBB, 1, TB)
    wflat = W.reshape(C).astype(jnp.float32)
    b16 = jnp.broadcast_to(b.reshape(()).astype(jnp.float32), (L,))
    b11 = b.reshape(1, 1).astype(jnp.float32)
    return _pooling(features, batch3, batchb, W.astype(jnp.float32), b11,
                    wflat, b16)


# speedup vs baseline: 1.7092x; 1.6849x over previous
import functools

import jax
import jax.numpy as jnp
from jax import lax
from jax.experimental import pallas as pl
from jax.experimental.pallas import tpu as pltpu
from jax.experimental.pallas import tpu_sc as plsc

N = 100000
C = 128
G = 64
NC = 2
NS = 16
NW = NC * NS
K = 160
KH = K // 2
TCH = N // K
SCH = 375
FULL = SCH % NW
CPW = SCH // NW
L = 16
CJ = C // L
U = 4

TB = 2000
TC0 = SCH * K // TB
TGRID = (N - SCH * K) // TB
NBB = N // TB


def _sc_body(feat, batch3, wflat, b16, out, acc, idx_v, idx_s, fbuf, sbuf,
             wv, bv, z8, semf, semi, sems):
    cid = lax.axis_index("c")
    sid = lax.axis_index("s")
    wid = cid * NS + sid

    pltpu.sync_copy(wflat, wv)
    pltpu.sync_copy(b16, bv)

    zero = jnp.zeros((L,), jnp.float32)
    for r in range(8):
        for j in range(CJ):
            z8[r, pl.ds(L * j, L)] = zero

    @pl.when(sid < 8)
    def _():
        pltpu.sync_copy(z8, acc.at[pl.ds(sid * 8, 8)])

    plsc.subcore_barrier()

    bvec = bv[...]
    wregs = [wv[pl.ds(L * j, L)] for j in range(CJ)]
    nch = jnp.where(wid < FULL, CPW + 1, CPW)
    start = wid * CPW + jnp.minimum(wid, FULL)

    lanes = lax.iota(jnp.int32, L)
    bfly = [lanes ^ sh for sh in (8, 4, 2, 1)]
    dnums = lax.GatherDimensionNumbers(
        offset_dims=(), collapsed_slice_dims=(0,), start_index_map=(0,)
    )

    def take16(x, idx):
        return lax.gather(
            x,
            idx[:, None],
            dnums,
            slice_sizes=(1,),
            mode=lax.GatherScatterMode.PROMISE_IN_BOUNDS,
        )

    def lane_sum_splat(x):
        for idx in bfly:
            x = x + take16(x, idx)
        return x

    def issue_fetch(c, p):
        pltpu.async_copy(feat.at[pl.ds(c * K, K)], fbuf.at[p], semf.at[p])
        pltpu.async_copy(batch3.at[c], idx_v.at[p], semi.at[p])

    def wait_fetch(p):
        pltpu.make_async_copy(feat.at[pl.ds(0, K)], fbuf.at[p], semf.at[p]).wait()
        pltpu.make_async_copy(batch3.at[0], idx_v.at[p], semi.at[p]).wait()

    def issue_scatter(p):
        for h in range(2):
            pltpu.async_copy(
                sbuf.at[p, pl.ds(h * KH, KH)],
                acc.at[idx_s.at[p, h]],
                sems.at[p],
                add=True,
            )

    def wait_scatter(p):
        for h in range(2):
            pltpu.make_async_copy(
                sbuf.at[p, pl.ds(h * KH, KH)], acc.at[idx_s.at[p, h]], sems.at[p]
            ).wait()

    def process(c, t, p):
        wait_fetch(p)

        @pl.when(t >= 2)
        def _():
            wait_scatter(p)

        for v in range(K // L):
            idx_s[p, v // (KH // L), pl.ds((v % (KH // L)) * L, L)] = idx_v[
                p, v // (KH // L), pl.ds((v % (KH // L)) * L, L)
            ]

        fb = fbuf.at[p]
        sb = sbuf.at[p]

        def scale_row(r):
            fr = [fb[r, pl.ds(L * j, L)] for j in range(CJ)]
            m = [fr[j] * wregs[j] for j in range(CJ)]
            while len(m) > 1:
                m = [m[2 * i] + m[2 * i + 1] for i in range(len(m) // 2)]
            s = lane_sum_splat(m[0])
            t_ = bvec + s
            wgt = 1.0 / (1.0 + jnp.exp(-t_))
            for j in range(CJ):
                sb[r, pl.ds(L * j, L)] = fr[j] * wgt

        def row_body(rq, rc):
            for i in range(U):
                scale_row(rq * U + i)
            return rc

        lax.fori_loop(0, K // U, row_body, 0)
        issue_scatter(p)

        @pl.when(t + 2 < nch)
        def _():
            issue_fetch(c + 2, p)

    issue_fetch(start, 0)
    issue_fetch(start + 1, 1)

    def chunk_body(t, carry):
        c = start + t

        @pl.when(t % 2 == 0)
        def _():
            process(c, t, 0)

        @pl.when(t % 2 == 1)
        def _():
            process(c, t, 1)

        return carry

    lax.fori_loop(0, nch, chunk_body, 0)
    wait_scatter(0)
    wait_scatter(1)
    plsc.subcore_barrier()

    @pl.when(sid == 0)
    def _():
        pltpu.sync_copy(acc, out.at[cid])


def _tc_body(feat_ref, batch_ref, w_ref, b_ref, out_ref):
    i = pl.program_id(0)
    blk = feat_ref[...]
    s = jnp.dot(blk, w_ref[...], preferred_element_type=jnp.float32)
    wgt = jax.nn.sigmoid(s + b_ref[0, 0])
    weighted = blk * wgt
    ids = batch_ref[0, 0, :]
    seg = lax.broadcasted_iota(jnp.int32, (G, TB), 0)
    onehot = (ids[None, :] == seg).astype(jnp.float32)
    part = jnp.dot(onehot, weighted, preferred_element_type=jnp.float32)

    @pl.when(i == 0)
    def _():
        out_ref[...] = part

    @pl.when(i > 0)
    def _():
        out_ref[...] += part


@jax.jit
def _pooling(features, batch3, batchb, wmat, b11, wflat, b16):
    mesh = plsc.VectorSubcoreMesh(core_axis_name="c", subcore_axis_name="s")
    sc_fn = functools.partial(
        pl.kernel,
        mesh=mesh,
        out_type=jax.ShapeDtypeStruct((NC, G, C), jnp.float32),
        scratch_types=[
            pltpu.VMEM_SHARED((G, C), jnp.float32),
            pltpu.VMEM((2, 2, KH), jnp.int32),
            pltpu.VMEM((2, 2, KH), jnp.int32),
            pltpu.VMEM((2, K, C), jnp.float32),
            pltpu.VMEM((2, K, C), jnp.float32),
            pltpu.VMEM((C,), jnp.float32),
            pltpu.VMEM((L,), jnp.float32),
            pltpu.VMEM((8, C), jnp.float32),
            pltpu.SemaphoreType.DMA((2,)),
            pltpu.SemaphoreType.DMA((2,)),
            pltpu.SemaphoreType.DMA((2,)),
        ],
    )(_sc_body)
    sc_part = sc_fn(features, batch3, wflat, b16)

    tc_part = pl.pallas_call(
        _tc_body,
        grid=(TGRID,),
        in_specs=[
            pl.BlockSpec((TB, C), lambda i: (TC0 + i, 0)),
            pl.BlockSpec((1, 1, TB), lambda i: (TC0 + i, 0, 0)),
            pl.BlockSpec((C, 1), lambda i: (0, 0)),
            pl.BlockSpec((1, 1), lambda i: (0, 0)),
        ],
        out_specs=pl.BlockSpec((G, C), lambda i: (0, 0)),
        out_shape=jax.ShapeDtypeStruct((G, C), jnp.float32),
    )(features, batchb, wmat, b11)

    return sc_part[0] + sc_part[1] + tc_part


def kernel(features, batch, W, b):
    bi = batch.astype(jnp.int32)
    batch3 = bi.reshape(TCH, 2, KH)
    batchb = bi.reshape(NBB, 1, TB)
    wflat = W.reshape(C).astype(jnp.float32)
    b16 = jnp.broadcast_to(b.reshape(()).astype(jnp.float32), (L,))
    b11 = b.reshape(1, 1).astype(jnp.float32)
    return _pooling(features, batch3, batchb, W.astype(jnp.float32), b11,
                    wflat, b16)


# speedup vs baseline: 1.7472x; 1.0222x over previous
import functools

import jax
import jax.numpy as jnp
from jax import lax
from jax.experimental import pallas as pl
from jax.experimental.pallas import tpu as pltpu
from jax.experimental.pallas import tpu_sc as plsc

N = 100000
C = 128
G = 64
NC = 2
NS = 16
NW = NC * NS
K = 160
KH = K // 2
TCH = N // K
SCH = 325
FULL = SCH % NW
CPW = SCH // NW
L = 16
CJ = C // L
U = 4

TB = 4000
TC0 = SCH * K // TB
TGRID = (N - SCH * K) // TB
NBB = N // TB


def _sc_body(feat, batch3, wflat, b16, out, acc, idx_v, idx_s, fbuf, sbuf,
             wv, bv, z8, semf, semi, sems):
    cid = lax.axis_index("c")
    sid = lax.axis_index("s")
    wid = cid * NS + sid

    pltpu.sync_copy(wflat, wv)
    pltpu.sync_copy(b16, bv)

    zero = jnp.zeros((L,), jnp.float32)
    for r in range(8):
        for j in range(CJ):
            z8[r, pl.ds(L * j, L)] = zero

    @pl.when(sid < 8)
    def _():
        pltpu.sync_copy(z8, acc.at[pl.ds(sid * 8, 8)])

    plsc.subcore_barrier()

    bvec = bv[...]
    wregs = [wv[pl.ds(L * j, L)] for j in range(CJ)]
    nch = jnp.where(wid < FULL, CPW + 1, CPW)
    start = wid * CPW + jnp.minimum(wid, FULL)

    lanes = lax.iota(jnp.int32, L)
    bfly = [lanes ^ sh for sh in (8, 4, 2, 1)]
    dnums = lax.GatherDimensionNumbers(
        offset_dims=(), collapsed_slice_dims=(0,), start_index_map=(0,)
    )

    def take16(x, idx):
        return lax.gather(
            x,
            idx[:, None],
            dnums,
            slice_sizes=(1,),
            mode=lax.GatherScatterMode.PROMISE_IN_BOUNDS,
        )

    def lane_sum_splat(x):
        for idx in bfly:
            x = x + take16(x, idx)
        return x

    def issue_fetch(c, p):
        pltpu.async_copy(feat.at[pl.ds(c * K, K)], fbuf.at[p], semf.at[p])
        pltpu.async_copy(batch3.at[c], idx_v.at[p], semi.at[p])

    def wait_fetch(p):
        pltpu.make_async_copy(feat.at[pl.ds(0, K)], fbuf.at[p], semf.at[p]).wait()
        pltpu.make_async_copy(batch3.at[0], idx_v.at[p], semi.at[p]).wait()

    def issue_scatter(p):
        for h in range(2):
            pltpu.async_copy(
                sbuf.at[p, pl.ds(h * KH, KH)],
                acc.at[idx_s.at[p, h]],
                sems.at[p],
                add=True,
            )

    def wait_scatter(p):
        for h in range(2):
            pltpu.make_async_copy(
                sbuf.at[p, pl.ds(h * KH, KH)], acc.at[idx_s.at[p, h]], sems.at[p]
            ).wait()

    def process(c, t, p):
        wait_fetch(p)

        @pl.when(t >= 2)
        def _():
            wait_scatter(p)

        for v in range(K // L):
            idx_s[p, v // (KH // L), pl.ds((v % (KH // L)) * L, L)] = idx_v[
                p, v // (KH // L), pl.ds((v % (KH // L)) * L, L)
            ]

        fb = fbuf.at[p]
        sb = sbuf.at[p]

        def scale_row(r):
            fr = [fb[r, pl.ds(L * j, L)] for j in range(CJ)]
            m = [fr[j] * wregs[j] for j in range(CJ)]
            while len(m) > 1:
                m = [m[2 * i] + m[2 * i + 1] for i in range(len(m) // 2)]
            s = lane_sum_splat(m[0])
            t_ = bvec + s
            wgt = 1.0 / (1.0 + jnp.exp(-t_))
            for j in range(CJ):
                sb[r, pl.ds(L * j, L)] = fr[j] * wgt

        def row_body(rq, rc):
            for i in range(U):
                scale_row(rq * U + i)
            return rc

        lax.fori_loop(0, K // U, row_body, 0)
        issue_scatter(p)

        @pl.when(t + 2 < nch)
        def _():
            issue_fetch(c + 2, p)

    issue_fetch(start, 0)
    issue_fetch(start + 1, 1)

    def chunk_body(t, carry):
        c = start + t

        @pl.when(t % 2 == 0)
        def _():
            process(c, t, 0)

        @pl.when(t % 2 == 1)
        def _():
            process(c, t, 1)

        return carry

    lax.fori_loop(0, nch, chunk_body, 0)
    wait_scatter(0)
    wait_scatter(1)
    plsc.subcore_barrier()

    @pl.when(sid == 0)
    def _():
        pltpu.sync_copy(acc, out.at[cid])


def _tc_body(feat_ref, batch_ref, w_ref, b_ref, out_ref):
    i = pl.program_id(0)
    blk = feat_ref[...]
    s = jnp.dot(blk, w_ref[...], preferred_element_type=jnp.float32)
    wgt = jax.nn.sigmoid(s + b_ref[0, 0])
    weighted = blk * wgt
    ids = batch_ref[0, 0, :]
    seg = lax.broadcasted_iota(jnp.int32, (G, TB), 0)
    onehot = (ids[None, :] == seg).astype(jnp.float32)
    part = jnp.dot(onehot, weighted, preferred_element_type=jnp.float32)

    @pl.when(i == 0)
    def _():
        out_ref[...] = part

    @pl.when(i > 0)
    def _():
        out_ref[...] += part


@jax.jit
def _pooling(features, batch3, batchb, wmat, b11, wflat, b16):
    mesh = plsc.VectorSubcoreMesh(core_axis_name="c", subcore_axis_name="s")
    sc_fn = functools.partial(
        pl.kernel,
        mesh=mesh,
        out_type=jax.ShapeDtypeStruct((NC, G, C), jnp.float32),
        scratch_types=[
            pltpu.VMEM_SHARED((G, C), jnp.float32),
            pltpu.VMEM((2, 2, KH), jnp.int32),
            pltpu.VMEM((2, 2, KH), jnp.int32),
            pltpu.VMEM((2, K, C), jnp.float32),
            pltpu.VMEM((2, K, C), jnp.float32),
            pltpu.VMEM((C,), jnp.float32),
            pltpu.VMEM((L,), jnp.float32),
            pltpu.VMEM((8, C), jnp.float32),
            pltpu.SemaphoreType.DMA((2,)),
            pltpu.SemaphoreType.DMA((2,)),
            pltpu.SemaphoreType.DMA((2,)),
        ],
    )(_sc_body)
    sc_part = sc_fn(features, batch3, wflat, b16)

    tc_part = pl.pallas_call(
        _tc_body,
        grid=(TGRID,),
        in_specs=[
            pl.BlockSpec((TB, C), lambda i: (TC0 + i, 0)),
            pl.BlockSpec((1, 1, TB), lambda i: (TC0 + i, 0, 0)),
            pl.BlockSpec((C, 1), lambda i: (0, 0)),
            pl.BlockSpec((1, 1), lambda i: (0, 0)),
        ],
        out_specs=pl.BlockSpec((G, C), lambda i: (0, 0)),
        out_shape=jax.ShapeDtypeStruct((G, C), jnp.float32),
    )(features, batchb, wmat, b11)

    return sc_part[0] + sc_part[1] + tc_part


def kernel(features, batch, W, b):
    bi = batch.astype(jnp.int32)
    batch3 = bi.reshape(TCH, 2, KH)
    batchb = bi.reshape(NBB, 1, TB)
    wflat = W.reshape(C).astype(jnp.float32)
    b16 = jnp.broadcast_to(b.reshape(()).astype(jnp.float32), (L,))
    b11 = b.reshape(1, 1).astype(jnp.float32)
    return _pooling(features, batch3, batchb, W.astype(jnp.float32), b11,
                    wflat, b16)
